# baseline (speedup 1.0000x reference)
import functools

import jax
import jax.numpy as jnp
from jax import lax
from jax.experimental import pallas as pl
from jax.experimental.pallas import tpu as pltpu
from jax.experimental.pallas import tpu_sc as plsc

NC, NS, LANES = 2, 16, 16
NW = NC * NS
BATCH = 128
AGG_FRAC0 = 0.66
DEGW = 128


def _fill_f32(ref, nrows, ncols, value):
    v = jnp.full((LANES,), value, jnp.float32)

    def row(i, carry):
        for ch in range(ncols // LANES):
            ref[i, pl.ds(ch * LANES, LANES)] = v
        return carry

    lax.fori_loop(0, nrows, row, 0)


def _zero_acc_slice(zero_v, acc, s, rows_per_tile, chunk):
    base = s * rows_per_tile
    off = 0
    while off < rows_per_tile:
        size = min(chunk, rows_per_tile - off)
        pltpu.sync_copy(zero_v.at[pl.ds(0, size)], acc.at[pl.ds(base + off, size)])
        off += size


def _sc_mesh():
    return plsc.VectorSubcoreMesh(core_axis_name="c", subcore_axis_name="s",
                                  num_cores=NC, num_subcores=NS)


def _acc_rows(n):
    per_tile = (n + NS) // NS
    rpt = ((per_tile + 7) // 8) * 8
    return NS * rpt, rpt


def _make_deg_kernel(n, nb):
    n_acc, rpt = _acc_rows(n)

    @functools.partial(
        pl.kernel,
        out_type=jax.ShapeDtypeStruct((NC, n_acc, DEGW), jnp.float32),
        mesh=_sc_mesh(),
        scratch_types=[
            pltpu.VMEM((nb, BATCH), jnp.int32),
            pltpu.VMEM((BATCH, DEGW), jnp.float32),
            pltpu.VMEM((BATCH, DEGW), jnp.float32),
            pltpu.VMEM_SHARED((n_acc, DEGW), jnp.float32),
        ],
    )
    def deg_kernel(dst_hbm, out_hbm, idx_v, ones_v, zero_v, acc):
        c = lax.axis_index("c")
        s = lax.axis_index("s")
        w = c * NS + s
        pltpu.sync_copy(dst_hbm.at[w], idx_v)
        _fill_f32(ones_v, BATCH, DEGW, 1.0)
        _fill_f32(zero_v, BATCH, DEGW, 0.0)
        _zero_acc_slice(zero_v, acc, s, rpt, BATCH)
        plsc.subcore_barrier()

        def step(j, carry):
            pltpu.sync_copy(ones_v, acc.at[idx_v.at[j]], add=True)
            return carry

        lax.fori_loop(0, nb, step, 0)
        plsc.subcore_barrier()
        ob = s * rpt
        pltpu.sync_copy(acc.at[pl.ds(ob, rpt)],
                        out_hbm.at[c, pl.ds(ob, rpt)])

    return deg_kernel


def _make_agg_kernel(n, h, nb0, nb1):
    n_acc, rpt = _acc_rows(n)
    nb_max = max(nb0, nb1)

    @functools.partial(
        pl.kernel,
        out_type=jax.ShapeDtypeStruct((NC, n_acc, h), jnp.float32),
        mesh=_sc_mesh(),
        scratch_types=[
            pltpu.VMEM((nb_max, BATCH), jnp.int32),
            pltpu.VMEM((nb_max, BATCH), jnp.int32),
            pltpu.VMEM((BATCH, h), jnp.float32),
            pltpu.VMEM((LANES, h), jnp.float32),
            pltpu.VMEM_SHARED((n_acc, h), jnp.float32),
            pltpu.SemaphoreType.DMA,
            pltpu.SemaphoreType.DMA,
        ],
    )
    def agg_kernel(tbl_hbm, src0_hbm, dst0_hbm, src1_hbm, dst1_hbm, out_hbm,
                   sidx, didx, rows, zero_v, acc, sem_a, sem_b):
        c = lax.axis_index("c")
        s = lax.axis_index("s")
        _fill_f32(zero_v, LANES, h, 0.0)
        _zero_acc_slice(zero_v, acc, s, rpt, LANES)
        plsc.subcore_barrier()
        tbl_c = tbl_hbm.at[c]
        hb = BATCH // 2

        def step(j, carry):
            row_j = sidx.at[j]
            ca = pltpu.async_copy(tbl_c.at[row_j.at[pl.ds(0, hb)]],
                                  rows.at[pl.ds(0, hb)], sem_a)
            cb = pltpu.async_copy(tbl_c.at[row_j.at[pl.ds(hb, hb)]],
                                  rows.at[pl.ds(hb, hb)], sem_b)
            ca.wait()
            cb.wait()
            pltpu.sync_copy(rows, acc.at[didx.at[j]], add=True)
            return carry

        @pl.when(c == 0)
        def _():
            pltpu.sync_copy(src0_hbm.at[s], sidx.at[pl.ds(0, nb0)])
            pltpu.sync_copy(dst0_hbm.at[s], didx.at[pl.ds(0, nb0)])
            lax.fori_loop(0, nb0, step, 0)

        @pl.when(c == 1)
        def _():
            pltpu.sync_copy(src1_hbm.at[s], sidx.at[pl.ds(0, nb1)])
            pltpu.sync_copy(dst1_hbm.at[s], didx.at[pl.ds(0, nb1)])
            lax.fori_loop(0, nb1, step, 0)
        plsc.subcore_barrier()
        ob = s * rpt
        pltpu.sync_copy(acc.at[pl.ds(ob, rpt)],
                        out_hbm.at[c, pl.ds(ob, rpt)])

    return agg_kernel


def _dinv_from_partials(degp_ref, n):
    deg = degp_ref[0, :n, 0:1] + degp_ref[1, :n, 0:1] + 1.0
    return lax.rsqrt(deg)


def _tc_matmul(x, W1):
    n = x.shape[0]
    hdim = W1.shape[0]

    def body(x_ref, w_ref, m_ref):
        m_ref[...] = lax.dot_general(x_ref[...], w_ref[...],
                                     (((1,), (1,)), ((), ())),
                                     preferred_element_type=jnp.float32)

    return pl.pallas_call(
        body, out_shape=jax.ShapeDtypeStruct((n, hdim), jnp.float32),
    )(x, W1)


def _tc_scale_dup(degp, m1):
    n, hdim = m1.shape

    def body(degp_ref, m_ref, h_ref):
        dinv = _dinv_from_partials(degp_ref, n)
        hv = m_ref[...] * dinv
        h_ref[0] = hv
        h_ref[1] = hv

    return pl.pallas_call(
        body, out_shape=jax.ShapeDtypeStruct((NC, n, hdim), jnp.float32),
    )(degp, m1)


def _layer_out(degp_ref, p_ref, h_ref, b_ref, g_ref, bt_ref, n):
    dinv = _dinv_from_partials(degp_ref, n)
    agg = p_ref[0, :n] + p_ref[1, :n] + h_ref[0]
    pre = dinv * agg + b_ref[...][None, :]
    r = jnp.maximum(pre, 0.0)
    mu = jnp.mean(r, axis=0, keepdims=True)
    var = jnp.mean((r - mu) ** 2, axis=0, keepdims=True)
    outl = (g_ref[...][None, :] * (r - mu) * lax.rsqrt(var + 1e-5)
            + bt_ref[...][None, :])
    return outl, dinv


def _tc_mid(degp, part, hpre, b, g, bt, Wn):
    n, hdim = hpre.shape[1:]

    def body(degp_ref, p_ref, h_ref, b_ref, g_ref, bt_ref, w_ref,
             out_ref, hn_ref):
        outl, dinv = _layer_out(degp_ref, p_ref, h_ref, b_ref, g_ref, bt_ref, n)
        out_ref[...] = outl
        hv = lax.dot_general(
            outl, w_ref[...], (((1,), (1,)), ((), ())),
            preferred_element_type=jnp.float32) * dinv
        hn_ref[0] = hv
        hn_ref[1] = hv

    return pl.pallas_call(
        body, out_shape=(jax.ShapeDtypeStruct((n, hdim), jnp.float32),
                         jax.ShapeDtypeStruct((NC, n, hdim), jnp.float32)),
    )(degp, part, hpre, b, g, bt, Wn)


def _tc_last(degp, part, hpre, b, g, bt, out1, out2, Wl, bl):
    n, hdim = hpre.shape[1:]
    cdim = Wl.shape[0]

    def body(degp_ref, p_ref, h_ref, b_ref, g_ref, bt_ref,
             o1_ref, o2_ref, wl_ref, bl_ref, y_ref):
        out3, _ = _layer_out(degp_ref, p_ref, h_ref, b_ref, g_ref, bt_ref, n)
        wl = wl_ref[...]
        dn = (((1,), (1,)), ((), ()))
        y = lax.dot_general(o1_ref[...], wl[:, :hdim], dn,
                            preferred_element_type=jnp.float32)
        y += lax.dot_general(o2_ref[...], wl[:, hdim:2 * hdim], dn,
                             preferred_element_type=jnp.float32)
        y += lax.dot_general(out3, wl[:, 2 * hdim:], dn,
                             preferred_element_type=jnp.float32)
        y_ref[...] = y + bl_ref[...][None, :]

    return pl.pallas_call(
        body, out_shape=jax.ShapeDtypeStruct((n, cdim), jnp.float32),
    )(degp, part, hpre, b, g, bt, out1, out2, Wl, bl)


def kernel(x, edge_index, W1, b1, g1, bt1, W2, b2, g2, bt2,
           W3, b3, g3, bt3, Wl, bl):
    n = x.shape[0]
    hdim = W1.shape[0]
    e = edge_index.shape[1]
    nb = -(-e // (NW * BATCH))
    e_pad = NW * nb * BATCH
    pad = e_pad - e
    dst = jnp.concatenate([edge_index[1], jnp.full((pad,), n, edge_index.dtype)])
    dst3 = dst.reshape(NW, nb, BATCH)

    nbt = -(-e // BATCH)
    nb0 = -(-int(nbt * AGG_FRAC0) // NS)
    nb1 = max(1, -(-max(nbt - NS * nb0, 0) // NS))
    cap = NS * (nb0 + nb1) * BATCH
    pad2 = cap - e
    srcp = jnp.concatenate([edge_index[0],
                            jnp.zeros((pad2,), edge_index.dtype)])
    dstp = jnp.concatenate([edge_index[1],
                            jnp.full((pad2,), n, edge_index.dtype)])
    e0 = NS * nb0 * BATCH
    src0 = srcp[:e0].reshape(NS, nb0, BATCH)
    dst0 = dstp[:e0].reshape(NS, nb0, BATCH)
    src1 = srcp[e0:].reshape(NS, nb1, BATCH)
    dst1 = dstp[e0:].reshape(NS, nb1, BATCH)

    degp = _make_deg_kernel(n, nb)(dst3)
    agg = _make_agg_kernel(n, hdim, nb0, nb1)

    m1 = _tc_matmul(x, W1)
    h1 = _tc_scale_dup(degp, m1)
    p1 = agg(h1, src0, dst0, src1, dst1)
    out1, h2 = _tc_mid(degp, p1, h1, b1, g1, bt1, W2)
    p2 = agg(h2, src0, dst0, src1, dst1)
    out2, h3 = _tc_mid(degp, p2, h2, b2, g2, bt2, W3)
    p3 = agg(h3, src0, dst0, src1, dst1)
    return _tc_last(degp, p3, h3, b3, g3, bt3, out1, out2, Wl, bl)


# speedup vs baseline: 1.0362x; 1.0362x over previous
import functools

import jax
import jax.numpy as jnp
from jax import lax
from jax.experimental import pallas as pl
from jax.experimental.pallas import tpu as pltpu
from jax.experimental.pallas import tpu_sc as plsc

NC, NS, LANES = 2, 16, 16
NW = NC * NS
BATCH = 128
AGG_FRAC0 = 0.62
DEGW = 128


def _fill_f32(ref, nrows, ncols, value):
    v = jnp.full((LANES,), value, jnp.float32)

    def row(i, carry):
        for ch in range(ncols // LANES):
            ref[i, pl.ds(ch * LANES, LANES)] = v
        return carry

    lax.fori_loop(0, nrows, row, 0)


def _zero_acc_slice(zero_v, acc, s, rows_per_tile, chunk):
    base = s * rows_per_tile
    off = 0
    while off < rows_per_tile:
        size = min(chunk, rows_per_tile - off)
        pltpu.sync_copy(zero_v.at[pl.ds(0, size)], acc.at[pl.ds(base + off, size)])
        off += size


def _sc_mesh():
    return plsc.VectorSubcoreMesh(core_axis_name="c", subcore_axis_name="s",
                                  num_cores=NC, num_subcores=NS)


def _acc_rows(n):
    per_tile = (n + NS) // NS
    rpt = ((per_tile + 7) // 8) * 8
    return NS * rpt, rpt


def _make_deg_kernel(n, nb):
    n_acc, rpt = _acc_rows(n)

    @functools.partial(
        pl.kernel,
        out_type=jax.ShapeDtypeStruct((NC, n_acc, DEGW), jnp.float32),
        mesh=_sc_mesh(),
        scratch_types=[
            pltpu.VMEM((nb, BATCH), jnp.int32),
            pltpu.VMEM((BATCH, DEGW), jnp.float32),
            pltpu.VMEM((BATCH, DEGW), jnp.float32),
            pltpu.VMEM_SHARED((n_acc, DEGW), jnp.float32),
        ],
    )
    def deg_kernel(dst_hbm, out_hbm, idx_v, ones_v, zero_v, acc):
        c = lax.axis_index("c")
        s = lax.axis_index("s")
        w = c * NS + s
        pltpu.sync_copy(dst_hbm.at[w], idx_v)
        _fill_f32(ones_v, BATCH, DEGW, 1.0)
        _fill_f32(zero_v, BATCH, DEGW, 0.0)
        _zero_acc_slice(zero_v, acc, s, rpt, BATCH)
        plsc.subcore_barrier()

        def step(j, carry):
            pltpu.sync_copy(ones_v, acc.at[idx_v.at[j]], add=True)
            return carry

        lax.fori_loop(0, nb, step, 0)
        plsc.subcore_barrier()
        ob = s * rpt
        pltpu.sync_copy(acc.at[pl.ds(ob, rpt)],
                        out_hbm.at[c, pl.ds(ob, rpt)])

    return deg_kernel


def _make_agg_kernel(n, h, nb0, nb1):
    n_acc, rpt = _acc_rows(n)
    nb_max = max(nb0, nb1)

    @functools.partial(
        pl.kernel,
        out_type=jax.ShapeDtypeStruct((NC, n_acc, h), jnp.float32),
        mesh=_sc_mesh(),
        scratch_types=[
            pltpu.VMEM((nb_max, BATCH), jnp.int32),
            pltpu.VMEM((nb_max, BATCH), jnp.int32),
            pltpu.VMEM((BATCH, h), jnp.float32),
            pltpu.VMEM((LANES, h), jnp.float32),
            pltpu.VMEM_SHARED((n_acc, h), jnp.float32),
            pltpu.SemaphoreType.DMA,
            pltpu.SemaphoreType.DMA,
        ],
    )
    def agg_kernel(tbl_hbm, src0_hbm, dst0_hbm, src1_hbm, dst1_hbm, out_hbm,
                   sidx, didx, rows, zero_v, acc, sem_a, sem_b):
        c = lax.axis_index("c")
        s = lax.axis_index("s")
        _fill_f32(zero_v, LANES, h, 0.0)
        _zero_acc_slice(zero_v, acc, s, rpt, LANES)
        plsc.subcore_barrier()
        tbl_c = tbl_hbm.at[c]
        hb = BATCH // 2

        def step(j, carry):
            row_j = sidx.at[j]
            ca = pltpu.async_copy(tbl_c.at[row_j.at[pl.ds(0, hb)]],
                                  rows.at[pl.ds(0, hb)], sem_a)
            cb = pltpu.async_copy(tbl_c.at[row_j.at[pl.ds(hb, hb)]],
                                  rows.at[pl.ds(hb, hb)], sem_b)
            ca.wait()
            cb.wait()
            pltpu.sync_copy(rows, acc.at[didx.at[j]], add=True)
            return carry

        @pl.when(c == 0)
        def _():
            pltpu.sync_copy(src0_hbm.at[s], sidx.at[pl.ds(0, nb0)])
            pltpu.sync_copy(dst0_hbm.at[s], didx.at[pl.ds(0, nb0)])
            lax.fori_loop(0, nb0, step, 0)

        @pl.when(c == 1)
        def _():
            pltpu.sync_copy(src1_hbm.at[s], sidx.at[pl.ds(0, nb1)])
            pltpu.sync_copy(dst1_hbm.at[s], didx.at[pl.ds(0, nb1)])
            lax.fori_loop(0, nb1, step, 0)
        plsc.subcore_barrier()
        ob = s * rpt
        pltpu.sync_copy(acc.at[pl.ds(ob, rpt)],
                        out_hbm.at[c, pl.ds(ob, rpt)])

    return agg_kernel


def _dinv_from_partials(degp_ref, n):
    deg = degp_ref[0, :n, 0:1] + degp_ref[1, :n, 0:1] + 1.0
    return lax.rsqrt(deg)


def _tc_matmul(x, W1):
    n = x.shape[0]
    hdim = W1.shape[0]

    def body(x_ref, w_ref, m_ref):
        m_ref[...] = lax.dot_general(x_ref[...], w_ref[...],
                                     (((1,), (1,)), ((), ())),
                                     preferred_element_type=jnp.float32)

    return pl.pallas_call(
        body, out_shape=jax.ShapeDtypeStruct((n, hdim), jnp.float32),
    )(x, W1)


def _tc_scale_dup(degp, m1):
    n, hdim = m1.shape

    def body(degp_ref, m_ref, h_ref):
        dinv = _dinv_from_partials(degp_ref, n)
        hv = m_ref[...] * dinv
        h_ref[0] = hv
        h_ref[1] = hv

    return pl.pallas_call(
        body, out_shape=jax.ShapeDtypeStruct((NC, n, hdim), jnp.float32),
    )(degp, m1)


def _layer_out(degp_ref, p_ref, h_ref, b_ref, g_ref, bt_ref, n):
    dinv = _dinv_from_partials(degp_ref, n)
    agg = p_ref[0, :n] + p_ref[1, :n] + h_ref[0]
    pre = dinv * agg + b_ref[...][None, :]
    r = jnp.maximum(pre, 0.0)
    mu = jnp.mean(r, axis=0, keepdims=True)
    var = jnp.mean((r - mu) ** 2, axis=0, keepdims=True)
    outl = (g_ref[...][None, :] * (r - mu) * lax.rsqrt(var + 1e-5)
            + bt_ref[...][None, :])
    return outl, dinv


def _tc_mid(degp, part, hpre, b, g, bt, Wn):
    n, hdim = hpre.shape[1:]

    def body(degp_ref, p_ref, h_ref, b_ref, g_ref, bt_ref, w_ref,
             out_ref, hn_ref):
        outl, dinv = _layer_out(degp_ref, p_ref, h_ref, b_ref, g_ref, bt_ref, n)
        out_ref[...] = outl
        hv = lax.dot_general(
            outl, w_ref[...], (((1,), (1,)), ((), ())),
            preferred_element_type=jnp.float32) * dinv
        hn_ref[0] = hv
        hn_ref[1] = hv

    return pl.pallas_call(
        body, out_shape=(jax.ShapeDtypeStruct((n, hdim), jnp.float32),
                         jax.ShapeDtypeStruct((NC, n, hdim), jnp.float32)),
    )(degp, part, hpre, b, g, bt, Wn)


def _tc_last(degp, part, hpre, b, g, bt, out1, out2, Wl, bl):
    n, hdim = hpre.shape[1:]
    cdim = Wl.shape[0]

    def body(degp_ref, p_ref, h_ref, b_ref, g_ref, bt_ref,
             o1_ref, o2_ref, wl_ref, bl_ref, y_ref):
        out3, _ = _layer_out(degp_ref, p_ref, h_ref, b_ref, g_ref, bt_ref, n)
        wl = wl_ref[...]
        dn = (((1,), (1,)), ((), ()))
        y = lax.dot_general(o1_ref[...], wl[:, :hdim], dn,
                            preferred_element_type=jnp.float32)
        y += lax.dot_general(o2_ref[...], wl[:, hdim:2 * hdim], dn,
                             preferred_element_type=jnp.float32)
        y += lax.dot_general(out3, wl[:, 2 * hdim:], dn,
                             preferred_element_type=jnp.float32)
        y_ref[...] = y + bl_ref[...][None, :]

    return pl.pallas_call(
        body, out_shape=jax.ShapeDtypeStruct((n, cdim), jnp.float32),
    )(degp, part, hpre, b, g, bt, out1, out2, Wl, bl)


def kernel(x, edge_index, W1, b1, g1, bt1, W2, b2, g2, bt2,
           W3, b3, g3, bt3, Wl, bl):
    n = x.shape[0]
    hdim = W1.shape[0]
    e = edge_index.shape[1]
    nb = -(-e // (NW * BATCH))
    e_pad = NW * nb * BATCH
    pad = e_pad - e
    dst = jnp.concatenate([edge_index[1], jnp.full((pad,), n, edge_index.dtype)])
    dst3 = dst.reshape(NW, nb, BATCH)

    nbt = -(-e // BATCH)
    nb0 = -(-int(nbt * AGG_FRAC0) // NS)
    nb1 = max(1, -(-max(nbt - NS * nb0, 0) // NS))
    cap = NS * (nb0 + nb1) * BATCH
    pad2 = cap - e
    srcp = jnp.concatenate([edge_index[0],
                            jnp.zeros((pad2,), edge_index.dtype)])
    dstp = jnp.concatenate([edge_index[1],
                            jnp.full((pad2,), n, edge_index.dtype)])
    e0 = NS * nb0 * BATCH
    src0 = srcp[:e0].reshape(NS, nb0, BATCH)
    dst0 = dstp[:e0].reshape(NS, nb0, BATCH)
    src1 = srcp[e0:].reshape(NS, nb1, BATCH)
    dst1 = dstp[e0:].reshape(NS, nb1, BATCH)

    degp = _make_deg_kernel(n, nb)(dst3)
    agg = _make_agg_kernel(n, hdim, nb0, nb1)

    m1 = _tc_matmul(x, W1)
    h1 = _tc_scale_dup(degp, m1)
    p1 = agg(h1, src0, dst0, src1, dst1)
    out1, h2 = _tc_mid(degp, p1, h1, b1, g1, bt1, W2)
    p2 = agg(h2, src0, dst0, src1, dst1)
    out2, h3 = _tc_mid(degp, p2, h2, b2, g2, bt2, W3)
    p3 = agg(h3, src0, dst0, src1, dst1)
    return _tc_last(degp, p3, h3, b3, g3, bt3, out1, out2, Wl, bl)


# speedup vs baseline: 1.0700x; 1.0326x over previous
import functools

import jax
import jax.numpy as jnp
from jax import lax
from jax.experimental import pallas as pl
from jax.experimental.pallas import tpu as pltpu
from jax.experimental.pallas import tpu_sc as plsc

NC, NS, LANES = 2, 16, 16
NW = NC * NS
BATCH = 128
AGG_FRAC0 = 0.60
DEGW = 128


def _fill_f32(ref, nrows, ncols, value):
    v = jnp.full((LANES,), value, jnp.float32)

    def row(i, carry):
        for ch in range(ncols // LANES):
            ref[i, pl.ds(ch * LANES, LANES)] = v
        return carry

    lax.fori_loop(0, nrows, row, 0)


def _zero_acc_slice(zero_v, acc, s, rows_per_tile, chunk):
    base = s * rows_per_tile
    off = 0
    while off < rows_per_tile:
        size = min(chunk, rows_per_tile - off)
        pltpu.sync_copy(zero_v.at[pl.ds(0, size)], acc.at[pl.ds(base + off, size)])
        off += size


def _sc_mesh():
    return plsc.VectorSubcoreMesh(core_axis_name="c", subcore_axis_name="s",
                                  num_cores=NC, num_subcores=NS)


def _acc_rows(n):
    per_tile = (n + NS) // NS
    rpt = ((per_tile + 7) // 8) * 8
    return NS * rpt, rpt


def _make_deg_kernel(n, nb):
    n_acc, rpt = _acc_rows(n)

    @functools.partial(
        pl.kernel,
        out_type=jax.ShapeDtypeStruct((NC, n_acc, DEGW), jnp.float32),
        mesh=_sc_mesh(),
        scratch_types=[
            pltpu.VMEM((nb, BATCH), jnp.int32),
            pltpu.VMEM((BATCH, DEGW), jnp.float32),
            pltpu.VMEM((BATCH, DEGW), jnp.float32),
            pltpu.VMEM_SHARED((n_acc, DEGW), jnp.float32),
        ],
    )
    def deg_kernel(dst_hbm, out_hbm, idx_v, ones_v, zero_v, acc):
        c = lax.axis_index("c")
        s = lax.axis_index("s")
        w = c * NS + s
        pltpu.sync_copy(dst_hbm.at[w], idx_v)
        _fill_f32(ones_v, BATCH, DEGW, 1.0)
        _fill_f32(zero_v, BATCH, DEGW, 0.0)
        _zero_acc_slice(zero_v, acc, s, rpt, BATCH)
        plsc.subcore_barrier()

        def step(j, carry):
            pltpu.sync_copy(ones_v, acc.at[idx_v.at[j]], add=True)
            return carry

        lax.fori_loop(0, nb, step, 0)
        plsc.subcore_barrier()
        ob = s * rpt
        pltpu.sync_copy(acc.at[pl.ds(ob, rpt)],
                        out_hbm.at[c, pl.ds(ob, rpt)])

    return deg_kernel


def _make_agg_kernel(n, h, nb0, nb1):
    n_acc, rpt = _acc_rows(n)
    nb_max = max(nb0, nb1)

    @functools.partial(
        pl.kernel,
        out_type=jax.ShapeDtypeStruct((NC, n_acc, h), jnp.float32),
        mesh=_sc_mesh(),
        scratch_types=[
            pltpu.VMEM((nb_max, BATCH), jnp.int32),
            pltpu.VMEM((nb_max, BATCH), jnp.int32),
            pltpu.VMEM((BATCH, h), jnp.float32),
            pltpu.VMEM((LANES, h), jnp.float32),
            pltpu.VMEM_SHARED((n_acc, h), jnp.float32),
            pltpu.SemaphoreType.DMA,
            pltpu.SemaphoreType.DMA,
        ],
    )
    def agg_kernel(tbl_hbm, src0_hbm, dst0_hbm, src1_hbm, dst1_hbm, out_hbm,
                   sidx, didx, rows, zero_v, acc, sem_a, sem_b):
        c = lax.axis_index("c")
        s = lax.axis_index("s")
        _fill_f32(zero_v, LANES, h, 0.0)
        _zero_acc_slice(zero_v, acc, s, rpt, LANES)
        plsc.subcore_barrier()
        tbl_c = tbl_hbm.at[c]
        hb = BATCH // 2

        def step(j, carry):
            row_j = sidx.at[j]
            ca = pltpu.async_copy(tbl_c.at[row_j.at[pl.ds(0, hb)]],
                                  rows.at[pl.ds(0, hb)], sem_a)
            cb = pltpu.async_copy(tbl_c.at[row_j.at[pl.ds(hb, hb)]],
                                  rows.at[pl.ds(hb, hb)], sem_b)
            ca.wait()
            cb.wait()
            pltpu.sync_copy(rows, acc.at[didx.at[j]], add=True)
            return carry

        @pl.when(c == 0)
        def _():
            pltpu.sync_copy(src0_hbm.at[s], sidx.at[pl.ds(0, nb0)])
            pltpu.sync_copy(dst0_hbm.at[s], didx.at[pl.ds(0, nb0)])
            lax.fori_loop(0, nb0, step, 0)

        @pl.when(c == 1)
        def _():
            pltpu.sync_copy(src1_hbm.at[s], sidx.at[pl.ds(0, nb1)])
            pltpu.sync_copy(dst1_hbm.at[s], didx.at[pl.ds(0, nb1)])
            lax.fori_loop(0, nb1, step, 0)
        plsc.subcore_barrier()
        ob = s * rpt
        pltpu.sync_copy(acc.at[pl.ds(ob, rpt)],
                        out_hbm.at[c, pl.ds(ob, rpt)])

    return agg_kernel


def _dinv_from_partials(degp_ref, n):
    deg = degp_ref[0, :n, 0:1] + degp_ref[1, :n, 0:1] + 1.0
    return lax.rsqrt(deg)


def _tc_matmul(x, W1):
    n = x.shape[0]
    hdim = W1.shape[0]

    def body(x_ref, w_ref, m_ref):
        m_ref[...] = lax.dot_general(x_ref[...], w_ref[...],
                                     (((1,), (1,)), ((), ())),
                                     preferred_element_type=jnp.float32)

    return pl.pallas_call(
        body, out_shape=jax.ShapeDtypeStruct((n, hdim), jnp.float32),
    )(x, W1)


def _tc_scale_dup(degp, m1):
    n, hdim = m1.shape

    def body(degp_ref, m_ref, h_ref):
        dinv = _dinv_from_partials(degp_ref, n)
        hv = m_ref[...] * dinv
        h_ref[0] = hv
        h_ref[1] = hv

    return pl.pallas_call(
        body, out_shape=jax.ShapeDtypeStruct((NC, n, hdim), jnp.float32),
    )(degp, m1)


def _layer_out(degp_ref, p_ref, h_ref, b_ref, g_ref, bt_ref, n):
    dinv = _dinv_from_partials(degp_ref, n)
    agg = p_ref[0, :n] + p_ref[1, :n] + h_ref[0]
    pre = dinv * agg + b_ref[...][None, :]
    r = jnp.maximum(pre, 0.0)
    mu = jnp.mean(r, axis=0, keepdims=True)
    var = jnp.mean((r - mu) ** 2, axis=0, keepdims=True)
    outl = (g_ref[...][None, :] * (r - mu) * lax.rsqrt(var + 1e-5)
            + bt_ref[...][None, :])
    return outl, dinv


def _tc_mid(degp, part, hpre, b, g, bt, Wn):
    n, hdim = hpre.shape[1:]

    def body(degp_ref, p_ref, h_ref, b_ref, g_ref, bt_ref, w_ref,
             out_ref, hn_ref):
        outl, dinv = _layer_out(degp_ref, p_ref, h_ref, b_ref, g_ref, bt_ref, n)
        out_ref[...] = outl
        hv = lax.dot_general(
            outl, w_ref[...], (((1,), (1,)), ((), ())),
            preferred_element_type=jnp.float32) * dinv
        hn_ref[0] = hv
        hn_ref[1] = hv

    return pl.pallas_call(
        body, out_shape=(jax.ShapeDtypeStruct((n, hdim), jnp.float32),
                         jax.ShapeDtypeStruct((NC, n, hdim), jnp.float32)),
    )(degp, part, hpre, b, g, bt, Wn)


def _tc_last(degp, part, hpre, b, g, bt, out1, out2, Wl, bl):
    n, hdim = hpre.shape[1:]
    cdim = Wl.shape[0]

    def body(degp_ref, p_ref, h_ref, b_ref, g_ref, bt_ref,
             o1_ref, o2_ref, wl_ref, bl_ref, y_ref):
        out3, _ = _layer_out(degp_ref, p_ref, h_ref, b_ref, g_ref, bt_ref, n)
        wl = wl_ref[...]
        dn = (((1,), (1,)), ((), ()))
        y = lax.dot_general(o1_ref[...], wl[:, :hdim], dn,
                            preferred_element_type=jnp.float32)
        y += lax.dot_general(o2_ref[...], wl[:, hdim:2 * hdim], dn,
                             preferred_element_type=jnp.float32)
        y += lax.dot_general(out3, wl[:, 2 * hdim:], dn,
                             preferred_element_type=jnp.float32)
        y_ref[...] = y + bl_ref[...][None, :]

    return pl.pallas_call(
        body, out_shape=jax.ShapeDtypeStruct((n, cdim), jnp.float32),
    )(degp, part, hpre, b, g, bt, out1, out2, Wl, bl)


def kernel(x, edge_index, W1, b1, g1, bt1, W2, b2, g2, bt2,
           W3, b3, g3, bt3, Wl, bl):
    n = x.shape[0]
    hdim = W1.shape[0]
    e = edge_index.shape[1]
    nb = -(-e // (NW * BATCH))
    e_pad = NW * nb * BATCH
    pad = e_pad - e
    dst = jnp.concatenate([edge_index[1], jnp.full((pad,), n, edge_index.dtype)])
    dst3 = dst.reshape(NW, nb, BATCH)

    nbt = -(-e // BATCH)
    nb0 = -(-int(nbt * AGG_FRAC0) // NS)
    nb1 = max(1, -(-max(nbt - NS * nb0, 0) // NS))
    cap = NS * (nb0 + nb1) * BATCH
    pad2 = cap - e
    srcp = jnp.concatenate([edge_index[0],
                            jnp.zeros((pad2,), edge_index.dtype)])
    dstp = jnp.concatenate([edge_index[1],
                            jnp.full((pad2,), n, edge_index.dtype)])
    e0 = NS * nb0 * BATCH
    src0 = srcp[:e0].reshape(NS, nb0, BATCH)
    dst0 = dstp[:e0].reshape(NS, nb0, BATCH)
    src1 = srcp[e0:].reshape(NS, nb1, BATCH)
    dst1 = dstp[e0:].reshape(NS, nb1, BATCH)

    degp = _make_deg_kernel(n, nb)(dst3)
    agg = _make_agg_kernel(n, hdim, nb0, nb1)

    m1 = _tc_matmul(x, W1)
    h1 = _tc_scale_dup(degp, m1)
    p1 = agg(h1, src0, dst0, src1, dst1)
    out1, h2 = _tc_mid(degp, p1, h1, b1, g1, bt1, W2)
    p2 = agg(h2, src0, dst0, src1, dst1)
    out2, h3 = _tc_mid(degp, p2, h2, b2, g2, bt2, W3)
    p3 = agg(h3, src0, dst0, src1, dst1)
    return _tc_last(degp, p3, h3, b3, g3, bt3, out1, out2, Wl, bl)
